# baseline (speedup 1.0000x reference)
import functools

import jax
import jax.numpy as jnp
from jax import lax
from jax.experimental import pallas as pl
from jax.experimental.pallas import tpu as pltpu
from jax.experimental.pallas import tpu_sc as plsc

N_CORES = 2
N_SUBCORES = 16
N_WORKERS = N_CORES * N_SUBCORES
CHUNK = 40


def _gemm_body(x_ref, w_ref, o_ref):
    o_ref[...] = lax.dot_general(
        x_ref[...], w_ref[...],
        dimension_numbers=(((1,), (1,)), ((), ())),
        preferred_element_type=jnp.float32,
    )


def _add_body(p_ref, o_ref):
    o_ref[...] = p_ref[0] + p_ref[1]


def _make_sc_aggregate(n_acc, n_edges, d):
    rows_per_tile = n_acc // N_SUBCORES
    edges_per_tile = n_edges // N_WORKERS
    n_chunks = edges_per_tile // CHUNK
    assert rows_per_tile * N_SUBCORES == n_acc and rows_per_tile % 8 == 0
    assert n_chunks * CHUNK == edges_per_tile

    mesh = plsc.VectorSubcoreMesh(core_axis_name="c", subcore_axis_name="s")
    NBUF = 8
    ZROWS = 32
    assert rows_per_tile % ZROWS == 0
    PEEL = next(p for p in range(1, NBUF + 1)
                if (n_chunks - NBUF + 1 - p) % NBUF == 0)
    assert n_chunks >= PEEL + 2 * NBUF

    @functools.partial(
        pl.kernel,
        out_type=jax.ShapeDtypeStruct((N_CORES, n_acc, d), jnp.float32),
        mesh=mesh,
        scratch_types=[
            [pltpu.VMEM((CHUNK,), jnp.int32)] * NBUF,
            [pltpu.VMEM((CHUNK,), jnp.int32)] * NBUF,
            [pltpu.VMEM((CHUNK, d), jnp.float32)] * NBUF,
            pltpu.VMEM((ZROWS, d), jnp.float32),
            pltpu.VMEM_SHARED((n_acc, d), jnp.float32),
            [pltpu.SemaphoreType.DMA] * NBUF,
            [pltpu.SemaphoreType.DMA] * NBUF,
            [pltpu.SemaphoreType.DMA] * NBUF,
        ],
    )
    def sc_aggregate(h_hbm, edge_hbm, out_hbm,
                     src_v, dst_v, rows, zbuf, acc, isem, gsem, ssem):
        c = lax.axis_index("c")
        s = lax.axis_index("s")
        base = (c * N_SUBCORES + s) * edges_per_tile
        r0 = s * rows_per_tile

        def start_idx(j, b):
            eb = base + j * CHUNK
            pltpu.async_copy(edge_hbm.at[pl.ds(eb, CHUNK)], src_v[b],
                             isem[b])
            pltpu.async_copy(edge_hbm.at[pl.ds(n_edges + eb, CHUNK)],
                             dst_v[b], isem[b])

        def wait_idx(j, b):
            eb = base + j * CHUNK
            pltpu.make_async_copy(edge_hbm.at[pl.ds(eb, CHUNK)], src_v[b],
                                  isem[b]).wait()
            pltpu.make_async_copy(edge_hbm.at[pl.ds(n_edges + eb, CHUNK)],
                                  dst_v[b], isem[b]).wait()

        def start_gather(b):
            pltpu.async_copy(h_hbm.at[src_v[b]], rows[b], gsem[b])

        def wait_gather(b):
            pltpu.make_async_copy(h_hbm.at[src_v[b]], rows[b], gsem[b]).wait()

        def start_scatter(b):
            pltpu.async_copy(rows[b], acc.at[dst_v[b]], ssem[b], add=True)

        def wait_scatter(b):
            pltpu.make_async_copy(rows[b], acc.at[dst_v[b]], ssem[b]).wait()

        for b in range(NBUF - 1):
            start_idx(b, b)
        zero16 = jnp.zeros((16,), jnp.float32)

        def zrow(i, _):
            for t in range(d // 16):
                zbuf[i, pl.ds(t * 16, 16)] = zero16
            return ()

        lax.fori_loop(0, ZROWS, zrow, ())
        for m in range(rows_per_tile // ZROWS):
            pltpu.sync_copy(zbuf, acc.at[pl.ds(r0 + m * ZROWS, ZROWS)])
        for b in range(NBUF - 2):
            wait_idx(b, b)
            start_gather(b)
        plsc.subcore_barrier()

        def step(j, b, wait_prev_scatter, do_idx, do_gather):
            bm1 = (b + NBUF - 1) % NBUF
            bm2 = (b + NBUF - 2) % NBUF
            wait_gather(b)
            start_scatter(b)
            if wait_prev_scatter:
                wait_scatter(bm1)
            if do_idx:
                start_idx(j + NBUF - 1, bm1)
            if do_gather:
                wait_idx(j + NBUF - 2, bm2)
                start_gather(bm2)

        for j in range(PEEL):
            step(j, j % NBUF, j > 0, True, True)

        def body(k, _):
            j0 = NBUF * k + PEEL
            for t in range(NBUF):
                step(j0 + t, (PEEL + t) % NBUF, True, True, True)
            return ()

        n_full = n_chunks - NBUF + 1 - PEEL
        lax.fori_loop(0, n_full // NBUF, body, ())
        j1 = n_chunks - NBUF + 1
        step(j1, j1 % NBUF, True, False, True)
        for j in range(j1 + 1, n_chunks):
            step(j, j % NBUF, True, False, False)
        wait_scatter((n_chunks - 1) % NBUF)
        plsc.subcore_barrier()
        pltpu.sync_copy(acc.at[pl.ds(r0, rows_per_tile)],
                        out_hbm.at[c, pl.ds(r0, rows_per_tile)])

    return sc_aggregate


def kernel(x, edge_index, weight):
    n_nodes, feat = x.shape
    embed = weight.shape[0]
    n_edges = edge_index.shape[1]

    bm = 2000
    h = pl.pallas_call(
        _gemm_body,
        grid=(n_nodes // bm,),
        in_specs=[
            pl.BlockSpec((bm, feat), lambda i: (i, 0)),
            pl.BlockSpec((embed, feat), lambda i: (0, 0)),
        ],
        out_specs=pl.BlockSpec((bm, embed), lambda i: (i, 0)),
        out_shape=jax.ShapeDtypeStruct((n_nodes, embed), jnp.float32),
    )(x, weight)

    pad = 64 * N_SUBCORES
    n_acc = ((n_nodes + pad - 1) // pad) * pad
    edge_flat = edge_index.reshape(2 * n_edges)
    partials = _make_sc_aggregate(n_acc, n_edges, embed)(h, edge_flat)

    out = pl.pallas_call(
        _add_body,
        grid=(n_nodes // bm,),
        in_specs=[pl.BlockSpec((N_CORES, bm, embed), lambda i: (0, i, 0))],
        out_specs=pl.BlockSpec((bm, embed), lambda i: (i, 0)),
        out_shape=jax.ShapeDtypeStruct((n_nodes, embed), jnp.float32),
    )(partials)
    return out


# speedup vs baseline: 1.0290x; 1.0290x over previous
import functools

import jax
import jax.numpy as jnp
from jax import lax
from jax.experimental import pallas as pl
from jax.experimental.pallas import tpu as pltpu
from jax.experimental.pallas import tpu_sc as plsc

N_CORES = 2
N_SUBCORES = 16
N_WORKERS = N_CORES * N_SUBCORES
CHUNK = 40


def _gemm_body(x_ref, w_ref, o_ref):
    o_ref[...] = lax.dot_general(
        x_ref[...], w_ref[...],
        dimension_numbers=(((1,), (1,)), ((), ())),
        preferred_element_type=jnp.float32,
    )


def _add_body(p_ref, o_ref):
    o_ref[...] = p_ref[0] + p_ref[1]


def _make_sc_aggregate(n_acc, n_edges, d):
    rows_per_tile = n_acc // N_SUBCORES
    edges_per_tile = n_edges // N_WORKERS
    n_chunks = edges_per_tile // CHUNK
    assert rows_per_tile * N_SUBCORES == n_acc and rows_per_tile % 8 == 0
    assert n_chunks * CHUNK == edges_per_tile

    mesh = plsc.VectorSubcoreMesh(core_axis_name="c", subcore_axis_name="s")
    NBUF = 8
    ZROWS = 32
    assert rows_per_tile % ZROWS == 0
    PEEL = next(p for p in range(1, NBUF + 1)
                if (n_chunks - NBUF + 1 - p) % NBUF == 0)
    assert n_chunks >= PEEL + 2 * NBUF

    @functools.partial(
        pl.kernel,
        out_type=jax.ShapeDtypeStruct((N_CORES, n_acc, d), jnp.float32),
        mesh=mesh,
        scratch_types=[
            [pltpu.VMEM((CHUNK,), jnp.int32)] * NBUF,
            [pltpu.VMEM((CHUNK,), jnp.int32)] * NBUF,
            [pltpu.VMEM((CHUNK, d), jnp.float32)] * NBUF,
            pltpu.VMEM((ZROWS, d), jnp.float32),
            pltpu.VMEM_SHARED((n_acc, d), jnp.float32),
            [pltpu.SemaphoreType.DMA] * NBUF,
            [pltpu.SemaphoreType.DMA] * NBUF,
            [pltpu.SemaphoreType.DMA] * NBUF,
        ],
    )
    def sc_aggregate(h_hbm, edge_hbm, out_hbm,
                     src_v, dst_v, rows, zbuf, acc, isem, gsem, ssem):
        c = lax.axis_index("c")
        s = lax.axis_index("s")
        base = (c * N_SUBCORES + s) * edges_per_tile
        r0 = s * rows_per_tile

        def start_idx(j, b):
            eb = base + j * CHUNK
            pltpu.async_copy(edge_hbm.at[pl.ds(eb, CHUNK)], src_v[b],
                             isem[b])
            pltpu.async_copy(edge_hbm.at[pl.ds(n_edges + eb, CHUNK)],
                             dst_v[b], isem[b])

        def wait_idx(j, b):
            eb = base + j * CHUNK
            pltpu.make_async_copy(edge_hbm.at[pl.ds(eb, CHUNK)], src_v[b],
                                  isem[b]).wait()
            pltpu.make_async_copy(edge_hbm.at[pl.ds(n_edges + eb, CHUNK)],
                                  dst_v[b], isem[b]).wait()

        def start_gather(b):
            pltpu.async_copy(h_hbm.at[src_v[b]], rows[b], gsem[b])

        def wait_gather(b):
            pltpu.make_async_copy(h_hbm.at[src_v[b]], rows[b], gsem[b]).wait()

        def start_scatter(b):
            pltpu.async_copy(rows[b], acc.at[dst_v[b]], ssem[b], add=True)

        def wait_scatter(b):
            pltpu.make_async_copy(rows[b], acc.at[dst_v[b]], ssem[b]).wait()

        for b in range(NBUF - 1):
            start_idx(b, b)
        zero16 = jnp.zeros((16,), jnp.float32)

        def zrow(i, _):
            for t in range(d // 16):
                zbuf[i, pl.ds(t * 16, 16)] = zero16
            return ()

        lax.fori_loop(0, ZROWS, zrow, ())
        for m in range(rows_per_tile // ZROWS):
            pltpu.sync_copy(zbuf, acc.at[pl.ds(r0 + m * ZROWS, ZROWS)])
        for b in range(NBUF - 2):
            wait_idx(b, b)
            start_gather(b)
        plsc.subcore_barrier()

        def step(j, b, wait_prev_scatter, do_idx, do_gather):
            bm1 = (b + NBUF - 1) % NBUF
            bm2 = (b + NBUF - 2) % NBUF
            wait_gather(b)
            start_scatter(b)
            if wait_prev_scatter:
                wait_scatter(bm1)
            if do_idx:
                start_idx(j + NBUF - 1, bm1)
            if do_gather:
                wait_idx(j + NBUF - 2, bm2)
                start_gather(bm2)

        for j in range(PEEL):
            step(j, j % NBUF, j > 0, True, True)

        def body(k, _):
            j0 = NBUF * k + PEEL
            for t in range(NBUF):
                step(j0 + t, (PEEL + t) % NBUF, True, True, True)
            return ()

        n_full = n_chunks - NBUF + 1 - PEEL
        lax.fori_loop(0, n_full // NBUF, body, ())
        j1 = n_chunks - NBUF + 1
        step(j1, j1 % NBUF, True, False, True)
        for j in range(j1 + 1, n_chunks):
            step(j, j % NBUF, True, False, False)
        wait_scatter((n_chunks - 1) % NBUF)
        plsc.subcore_barrier()
        pltpu.sync_copy(acc.at[pl.ds(r0, rows_per_tile)],
                        out_hbm.at[c, pl.ds(r0, rows_per_tile)])

    return sc_aggregate


def kernel(x, edge_index, weight):
    n_nodes, feat = x.shape
    embed = weight.shape[0]
    n_edges = edge_index.shape[1]

    bm = 5000
    h = pl.pallas_call(
        _gemm_body,
        grid=(n_nodes // bm,),
        in_specs=[
            pl.BlockSpec((bm, feat), lambda i: (i, 0)),
            pl.BlockSpec((embed, feat), lambda i: (0, 0)),
        ],
        out_specs=pl.BlockSpec((bm, embed), lambda i: (i, 0)),
        out_shape=jax.ShapeDtypeStruct((n_nodes, embed), jnp.float32),
    )(x, weight)

    pad = 64 * N_SUBCORES
    n_acc = ((n_nodes + pad - 1) // pad) * pad
    edge_flat = edge_index.reshape(2 * n_edges)
    partials = _make_sc_aggregate(n_acc, n_edges, embed)(h, edge_flat)

    out = pl.pallas_call(
        _add_body,
        grid=(n_nodes // bm,),
        in_specs=[pl.BlockSpec((N_CORES, bm, embed), lambda i: (0, i, 0))],
        out_specs=pl.BlockSpec((bm, embed), lambda i: (i, 0)),
        out_shape=jax.ShapeDtypeStruct((n_nodes, embed), jnp.float32),
    )(partials)
    return out
